# baseline (speedup 1.0000x reference)
import functools

import jax
import jax.numpy as jnp
from jax import lax
from jax.experimental import pallas as pl
from jax.experimental.pallas import tpu as pltpu
from jax.experimental.pallas import tpu_sc as plsc

N_NODES = 10000
N_EDGES = 320000
D = 128

NC = 2
NS = 16
NW = NC * NS

CHUNK = 128
SEG = 20
S0 = 6
S1 = 2
C0 = S0 * SEG
C1 = S1 * SEG
E_PAD = NS * (C0 + C1) * CHUNK
AGG_ROWS = N_NODES
STRIPE = 624
LAST_STRIPE = AGG_ROWS - 15 * STRIPE


def _sc_segment_sum(x, src_w, dst_w, zeros):
  mesh = plsc.VectorSubcoreMesh(core_axis_name="c", subcore_axis_name="s")

  @functools.partial(
      pl.kernel,
      out_type=jax.ShapeDtypeStruct((NC, AGG_ROWS, D), jnp.float32),
      mesh=mesh,
      scratch_types=[
          pltpu.VMEM((SEG, CHUNK), jnp.int32),
          pltpu.VMEM((SEG, CHUNK), jnp.int32),
          pltpu.VMEM((SEG, CHUNK), jnp.int32),
          pltpu.VMEM((SEG, CHUNK), jnp.int32),
          pltpu.VMEM((2, CHUNK, D), jnp.float32),
          pltpu.VMEM_SHARED((AGG_ROWS, D), jnp.float32),
          pltpu.SemaphoreType.DMA,
          pltpu.SemaphoreType.DMA,
          pltpu.SemaphoreType.DMA,
          pltpu.SemaphoreType.DMA,
      ],
  )
  def k(x_hbm, src_hbm, dst_hbm, zeros_hbm, out_hbm,
        sa_src, sa_dst, sb_src, sb_dst, rows_v, agg_sh, g0, g1, ia, ib):
    cid = lax.axis_index("c")
    sid = lax.axis_index("s")
    wid = cid * NS + sid
    nsegs = jnp.where(cid == 0, S0, S1)

    @pl.when(sid < NS - 1)
    def _():
      pltpu.sync_copy(zeros_hbm.at[pl.ds(sid * STRIPE, STRIPE)],
                      agg_sh.at[pl.ds(sid * STRIPE, STRIPE)])

    @pl.when(sid == NS - 1)
    def _():
      pltpu.sync_copy(zeros_hbm.at[pl.ds(15 * STRIPE, LAST_STRIPE)],
                      agg_sh.at[pl.ds(15 * STRIPE, LAST_STRIPE)])

    pltpu.sync_copy(src_hbm.at[wid, 0], sa_src)
    pltpu.sync_copy(dst_hbm.at[wid, 0], sa_dst)
    plsc.subcore_barrier()

    def gather(src_s, j, buf, sem):
      pltpu.async_copy(x_hbm.at[src_s.at[j]], rows_v.at[buf], sem)

    def gwait(buf, sem):
      pltpu.make_async_copy(x_hbm.at[sa_src.at[0]], rows_v.at[buf], sem).wait()

    def process_seg(src_s, dst_s):
      gather(src_s, 0, 0, g0)

      def body(jj, carry):
        j0 = jj * 2
        j1 = j0 + 1
        jpre = lax.rem(j0 + 2, SEG)
        gwait(0, g0)
        gather(src_s, j1, 1, g1)
        pltpu.sync_copy(rows_v.at[0], agg_sh.at[dst_s.at[j0]], add=True)
        gwait(1, g1)
        gather(src_s, jpre, 0, g0)
        pltpu.sync_copy(rows_v.at[1], agg_sh.at[dst_s.at[j1]], add=True)
        return carry

      lax.fori_loop(0, SEG // 2, body, 0, unroll=False)
      gwait(0, g0)

    def outer(ss, carry):
      s0 = ss * 2
      s1 = s0 + 1
      s2 = lax.rem(s0 + 2, nsegs)
      pltpu.async_copy(src_hbm.at[wid, s1], sb_src, ib)
      pltpu.async_copy(dst_hbm.at[wid, s1], sb_dst, ib)
      process_seg(sa_src, sa_dst)
      pltpu.make_async_copy(src_hbm.at[wid, 0], sb_src, ib).wait()
      pltpu.make_async_copy(dst_hbm.at[wid, 0], sb_dst, ib).wait()
      pltpu.async_copy(src_hbm.at[wid, s2], sa_src, ia)
      pltpu.async_copy(dst_hbm.at[wid, s2], sa_dst, ia)
      process_seg(sb_src, sb_dst)
      pltpu.make_async_copy(src_hbm.at[wid, 0], sa_src, ia).wait()
      pltpu.make_async_copy(dst_hbm.at[wid, 0], sa_dst, ia).wait()
      return carry

    lax.fori_loop(0, nsegs // 2, outer, 0, unroll=False)

    plsc.subcore_barrier()

    @pl.when(sid < NS - 1)
    def _():
      pltpu.sync_copy(agg_sh.at[pl.ds(sid * STRIPE, STRIPE)],
                      out_hbm.at[cid, pl.ds(sid * STRIPE, STRIPE)])

    @pl.when(sid == NS - 1)
    def _():
      pltpu.sync_copy(agg_sh.at[pl.ds(15 * STRIPE, LAST_STRIPE)],
                      out_hbm.at[cid, pl.ds(15 * STRIPE, LAST_STRIPE)])

  return k(x, src_w, dst_w, zeros)


def _tc_linear(x, p, w1t, w2t, b2):
  blk = 1000

  def body(x_ref, p_ref, w1_ref, w2_ref, b_ref, o_ref):
    agg = p_ref[0] + p_ref[1]
    o_ref[...] = (
        jnp.dot(x_ref[...], w1_ref[...], preferred_element_type=jnp.float32)
        + jnp.dot(agg, w2_ref[...], preferred_element_type=jnp.float32)
        + b_ref[...]
    )

  return pl.pallas_call(
      body,
      grid=(N_NODES // blk,),
      in_specs=[
          pl.BlockSpec((blk, D), lambda i: (i, 0)),
          pl.BlockSpec((NC, blk, D), lambda i: (0, i, 0)),
          pl.BlockSpec((D, D), lambda i: (0, 0)),
          pl.BlockSpec((D, D), lambda i: (0, 0)),
          pl.BlockSpec((1, D), lambda i: (0, 0)),
      ],
      out_specs=pl.BlockSpec((blk, D), lambda i: (i, 0)),
      out_shape=jax.ShapeDtypeStruct((N_NODES, D), jnp.float32),
  )(x, p, w1t, w2t, b2)


def _split_segments(a, fill):
  e0 = NS * C0 * CHUNK
  r0 = a[:e0].reshape(NS, S0, SEG, CHUNK)
  r1 = a[e0:].reshape(NS, S1, SEG, CHUNK)
  r1 = jnp.concatenate(
      [r1, jnp.full((NS, S0 - S1, SEG, CHUNK), fill, jnp.int32)], axis=1)
  return jnp.concatenate([r0, r1])


@jax.jit
def kernel(x, edge_index, W, b):
  pad = E_PAD - N_EDGES
  x_pad = jnp.concatenate([x, jnp.zeros((8, D), jnp.float32)])
  src = jnp.concatenate([edge_index[0], jnp.full((pad,), N_NODES, jnp.int32)])
  dst = jnp.concatenate([edge_index[1], jnp.zeros((pad,), jnp.int32)])
  src_w = _split_segments(src, N_NODES)
  dst_w = _split_segments(dst, 0)
  zeros = jnp.zeros((AGG_ROWS, D), jnp.float32)
  p = _sc_segment_sum(x_pad, src_w, dst_w, zeros)
  w1t = W[:, :D].T
  w2t = W[:, D:].T
  return _tc_linear(x, p, w1t, w2t, b.reshape(1, D))


# speedup vs baseline: 1.3415x; 1.3415x over previous
import functools

import jax
import jax.numpy as jnp
from jax import lax
from jax.experimental import pallas as pl
from jax.experimental.pallas import tpu as pltpu
from jax.experimental.pallas import tpu_sc as plsc

N_NODES = 10000
N_EDGES = 320000
D = 128

NC = 2
NS = 16
NW = NC * NS

CHUNK = 128
C0 = 117
C1 = 41
E_PAD = NS * (C0 + C1) * CHUNK
AGG_ROWS = 10008
STRIPE = 624
LAST_STRIPE = AGG_ROWS - 15 * STRIPE


def _sc_segment_sum(x, src_w, dst_w, zeros):
  mesh = plsc.VectorSubcoreMesh(core_axis_name="c", subcore_axis_name="s")

  @functools.partial(
      pl.kernel,
      out_type=jax.ShapeDtypeStruct((NC, AGG_ROWS, D), jnp.float32),
      mesh=mesh,
      scratch_types=[
          pltpu.VMEM((C0, CHUNK), jnp.int32),
          pltpu.VMEM((C0, CHUNK), jnp.int32),
          pltpu.VMEM((CHUNK, D), jnp.float32),
          pltpu.VMEM_SHARED((AGG_ROWS, D), jnp.float32),
          pltpu.SemaphoreType.DMA,
      ],
  )
  def k(x_hbm, src_hbm, dst_hbm, zeros_hbm, out_hbm,
        src_v, dst_v, rows_v, agg_sh, g0):
    cid = lax.axis_index("c")
    sid = lax.axis_index("s")
    wid = cid * NS + sid
    nchunks = jnp.where(cid == 0, C0, C1)

    @pl.when(sid < NS - 1)
    def _():
      pltpu.sync_copy(zeros_hbm.at[pl.ds(sid * STRIPE, STRIPE)],
                      agg_sh.at[pl.ds(sid * STRIPE, STRIPE)])

    @pl.when(sid == NS - 1)
    def _():
      pltpu.sync_copy(zeros_hbm.at[pl.ds(15 * STRIPE, LAST_STRIPE)],
                      agg_sh.at[pl.ds(15 * STRIPE, LAST_STRIPE)])

    pltpu.sync_copy(src_hbm.at[wid], src_v)
    pltpu.sync_copy(dst_hbm.at[wid], dst_v)
    plsc.subcore_barrier()

    def body(j, carry):
      pltpu.async_copy(x_hbm.at[src_v.at[j]], rows_v, g0).wait()
      pltpu.sync_copy(rows_v, agg_sh.at[dst_v.at[j]], add=True)
      return carry

    @pl.when(cid == 0)
    def _():
      lax.fori_loop(0, C0, body, 0, unroll=3)

    @pl.when(cid == 1)
    def _():
      lax.fori_loop(0, C1, body, 0, unroll=1)

    plsc.subcore_barrier()

    @pl.when(sid < NS - 1)
    def _():
      pltpu.sync_copy(agg_sh.at[pl.ds(sid * STRIPE, STRIPE)],
                      out_hbm.at[cid, pl.ds(sid * STRIPE, STRIPE)])

    @pl.when(sid == NS - 1)
    def _():
      pltpu.sync_copy(agg_sh.at[pl.ds(15 * STRIPE, LAST_STRIPE)],
                      out_hbm.at[cid, pl.ds(15 * STRIPE, LAST_STRIPE)])

  return k(x, src_w, dst_w, zeros)


def _tc_linear(x, p, w1t, w2t, b2):
  blk = 1000

  def body(x_ref, p_ref, w1_ref, w2_ref, b_ref, o_ref):
    agg = p_ref[0] + p_ref[1]
    o_ref[...] = (
        jnp.dot(x_ref[...], w1_ref[...], preferred_element_type=jnp.float32)
        + jnp.dot(agg, w2_ref[...], preferred_element_type=jnp.float32)
        + b_ref[...]
    )

  return pl.pallas_call(
      body,
      grid=(N_NODES // blk,),
      in_specs=[
          pl.BlockSpec((blk, D), lambda i: (i, 0)),
          pl.BlockSpec((NC, blk, D), lambda i: (0, i, 0)),
          pl.BlockSpec((D, D), lambda i: (0, 0)),
          pl.BlockSpec((D, D), lambda i: (0, 0)),
          pl.BlockSpec((1, D), lambda i: (0, 0)),
      ],
      out_specs=pl.BlockSpec((blk, D), lambda i: (i, 0)),
      out_shape=jax.ShapeDtypeStruct((N_NODES, D), jnp.float32),
  )(x, p, w1t, w2t, b2)


def _split_chunks(a, fill):
  e0 = NS * C0 * CHUNK
  r0 = a[:e0].reshape(NS, C0, CHUNK)
  r1 = a[e0:].reshape(NS, C1, CHUNK)
  r1 = jnp.concatenate(
      [r1, jnp.full((NS, C0 - C1, CHUNK), fill, jnp.int32)], axis=1)
  return jnp.concatenate([r0, r1])


@jax.jit
def kernel(x, edge_index, W, b):
  pad = E_PAD - N_EDGES
  src = jnp.concatenate([edge_index[0], jnp.zeros((pad,), jnp.int32)])
  dst = jnp.concatenate([edge_index[1], jnp.full((pad,), N_NODES, jnp.int32)])
  src_w = _split_chunks(src, 0)
  dst_w = _split_chunks(dst, N_NODES)
  zeros = jnp.zeros((AGG_ROWS, D), jnp.float32)
  p = _sc_segment_sum(x, src_w, dst_w, zeros)
  w1t = W[:, :D].T
  w2t = W[:, D:].T
  return _tc_linear(x, p, w1t, w2t, b.reshape(1, D))


# speedup vs baseline: 1.3823x; 1.0304x over previous
import functools

import jax
import jax.numpy as jnp
from jax import lax
from jax.experimental import pallas as pl
from jax.experimental.pallas import tpu as pltpu
from jax.experimental.pallas import tpu_sc as plsc

N_NODES = 10000
N_EDGES = 320000
D = 128

NC = 2
NS = 16
NW = NC * NS

CHUNK = 128
C0 = 117
C1 = 41
E_PAD = NS * (C0 + C1) * CHUNK
AGG_ROWS = 10008
STRIPE = 624
LAST_STRIPE = AGG_ROWS - 15 * STRIPE


def _sc_segment_sum(x, src0, src1, dst0, dst1, zeros):
  mesh = plsc.VectorSubcoreMesh(core_axis_name="c", subcore_axis_name="s")

  @functools.partial(
      pl.kernel,
      out_type=jax.ShapeDtypeStruct((NC, AGG_ROWS, D), jnp.float32),
      mesh=mesh,
      scratch_types=[
          pltpu.VMEM((C0, CHUNK), jnp.int32),
          pltpu.VMEM((C0, CHUNK), jnp.int32),
          pltpu.VMEM((CHUNK, D), jnp.float32),
          pltpu.VMEM_SHARED((AGG_ROWS, D), jnp.float32),
          pltpu.SemaphoreType.DMA,
      ],
  )
  def k(x_hbm, src0_hbm, src1_hbm, dst0_hbm, dst1_hbm, zeros_hbm, out_hbm,
        src_v, dst_v, rows_v, agg_sh, g0):
    cid = lax.axis_index("c")
    sid = lax.axis_index("s")

    @pl.when(sid < NS - 1)
    def _():
      pltpu.sync_copy(zeros_hbm.at[pl.ds(sid * STRIPE, STRIPE)],
                      agg_sh.at[pl.ds(sid * STRIPE, STRIPE)])

    @pl.when(sid == NS - 1)
    def _():
      pltpu.sync_copy(zeros_hbm.at[pl.ds(15 * STRIPE, LAST_STRIPE)],
                      agg_sh.at[pl.ds(15 * STRIPE, LAST_STRIPE)])

    @pl.when(cid == 0)
    def _():
      pltpu.sync_copy(src0_hbm.at[sid], src_v)
      pltpu.sync_copy(dst0_hbm.at[sid], dst_v)

    @pl.when(cid == 1)
    def _():
      pltpu.sync_copy(src1_hbm.at[sid], src_v.at[pl.ds(0, C1)])
      pltpu.sync_copy(dst1_hbm.at[sid], dst_v.at[pl.ds(0, C1)])

    plsc.subcore_barrier()

    def body(j, carry):
      pltpu.async_copy(x_hbm.at[src_v.at[j]], rows_v, g0).wait()
      pltpu.sync_copy(rows_v, agg_sh.at[dst_v.at[j]], add=True)
      return carry

    @pl.when(cid == 0)
    def _():
      lax.fori_loop(0, C0, body, 0, unroll=3)

    @pl.when(cid == 1)
    def _():
      lax.fori_loop(0, C1, body, 0, unroll=1)

    plsc.subcore_barrier()

    @pl.when(sid < NS - 1)
    def _():
      pltpu.sync_copy(agg_sh.at[pl.ds(sid * STRIPE, STRIPE)],
                      out_hbm.at[cid, pl.ds(sid * STRIPE, STRIPE)])

    @pl.when(sid == NS - 1)
    def _():
      pltpu.sync_copy(agg_sh.at[pl.ds(15 * STRIPE, LAST_STRIPE)],
                      out_hbm.at[cid, pl.ds(15 * STRIPE, LAST_STRIPE)])

  return k(x, src0, src1, dst0, dst1, zeros)


def _tc_linear(x, p, w1t, w2t, b2):
  blk = 1000

  def body(x_ref, p_ref, w1_ref, w2_ref, b_ref, o_ref):
    agg = p_ref[0] + p_ref[1]
    o_ref[...] = (
        jnp.dot(x_ref[...], w1_ref[...], preferred_element_type=jnp.float32)
        + jnp.dot(agg, w2_ref[...], preferred_element_type=jnp.float32)
        + b_ref[...]
    )

  return pl.pallas_call(
      body,
      grid=(N_NODES // blk,),
      in_specs=[
          pl.BlockSpec((blk, D), lambda i: (i, 0)),
          pl.BlockSpec((NC, blk, D), lambda i: (0, i, 0)),
          pl.BlockSpec((D, D), lambda i: (0, 0)),
          pl.BlockSpec((D, D), lambda i: (0, 0)),
          pl.BlockSpec((1, D), lambda i: (0, 0)),
      ],
      out_specs=pl.BlockSpec((blk, D), lambda i: (i, 0)),
      out_shape=jax.ShapeDtypeStruct((N_NODES, D), jnp.float32),
  )(x, p, w1t, w2t, b2)


@jax.jit
def kernel(x, edge_index, W, b):
  e0 = NS * C0 * CHUNK
  pad = E_PAD - N_EDGES
  src0 = edge_index[0, :e0].reshape(NS, C0, CHUNK)
  dst0 = edge_index[1, :e0].reshape(NS, C0, CHUNK)
  src1 = jnp.concatenate(
      [edge_index[0, e0:], jnp.zeros((pad,), jnp.int32)]
  ).reshape(NS, C1, CHUNK)
  dst1 = jnp.concatenate(
      [edge_index[1, e0:], jnp.full((pad,), N_NODES, jnp.int32)]
  ).reshape(NS, C1, CHUNK)
  zeros = jnp.zeros((AGG_ROWS, D), jnp.float32)
  p = _sc_segment_sum(x, src0, src1, dst0, dst1, zeros)
  w1t = W[:, :D].T
  w2t = W[:, D:].T
  return _tc_linear(x, p, w1t, w2t, b.reshape(1, D))
